# Initial kernel scaffold; baseline (speedup 1.0000x reference)
import jax
import jax.numpy as jnp
from jax import lax
from jax.experimental import pallas as pl
from jax.experimental.pallas import tpu as pltpu
from jax.experimental.pallas import tpu_sc as plsc

N = 10000
E = 320000
D = 128
R = 24

NC = 2
NS = 16
NW = NC * NS
EPW = E // NW
CH = 100
NCH = EPW // CH
STRIPE = N // NS
NBLK = 1000


def _proj_body(x_ref, w_ref, o_ref):
    r = pl.program_id(1)
    o_ref[0] = jnp.dot(x_ref[...], w_ref[r], preferred_element_type=jnp.float32)


def _project(x, W):
    return pl.pallas_call(
        _proj_body,
        grid=(N // NBLK, R),
        in_specs=[
            pl.BlockSpec((NBLK, D), lambda i, r: (i, 0)),
            pl.BlockSpec((R, D, D), lambda i, r: (0, 0, 0)),
        ],
        out_specs=pl.BlockSpec((1, NBLK, D), lambda i, r: (r, i, 0)),
        out_shape=jax.ShapeDtypeStruct((R, N, D), jnp.float32),
    )(x, W)


def _sc_body(xp_hbm, flat_hbm, dst_hbm, z128_hbm, z16_hbm, ones_hbm,
             msum_out, deg_out,
             gidx, sidx, rows0, rows1, ones_v, msum_sh, deg_sh, sem0, sem1):
    c = lax.axis_index("c")
    s = lax.axis_index("s")
    wid = c * NS + s

    for b in range(STRIPE // 125):
        off = s * STRIPE + b * 125
        pltpu.sync_copy(z128_hbm, msum_sh.at[pl.ds(off, 125)])
        pltpu.sync_copy(z16_hbm, deg_sh.at[pl.ds(off, 125)])

    pltpu.sync_copy(flat_hbm.at[wid], gidx)
    pltpu.sync_copy(dst_hbm.at[wid], sidx)
    pltpu.sync_copy(ones_hbm, ones_v)

    plsc.subcore_barrier()

    def pair(t, carry):
        j0 = t * 2
        j1 = j0 + 1
        cp0 = pltpu.async_copy(xp_hbm.at[gidx.at[j0]], rows0, sem0)
        cp1 = pltpu.async_copy(xp_hbm.at[gidx.at[j1]], rows1, sem1)
        cp0.wait()
        pltpu.sync_copy(rows0, msum_sh.at[sidx.at[j0]], add=True)
        pltpu.sync_copy(ones_v, deg_sh.at[sidx.at[j0]], add=True)
        cp1.wait()
        pltpu.sync_copy(rows1, msum_sh.at[sidx.at[j1]], add=True)
        pltpu.sync_copy(ones_v, deg_sh.at[sidx.at[j1]], add=True)
        return carry

    lax.fori_loop(0, NCH // 2, pair, 0)

    plsc.subcore_barrier()

    for b in range(STRIPE // 125):
        off = s * STRIPE + b * 125
        pltpu.sync_copy(msum_sh.at[pl.ds(off, 125)],
                        msum_out.at[c, pl.ds(off, 125)])
        pltpu.sync_copy(deg_sh.at[pl.ds(off, 125)],
                        deg_out.at[c, pl.ds(off, 125)])


def _sc_aggregate(xp, flat3, dst3):
    mesh = plsc.VectorSubcoreMesh(core_axis_name="c", subcore_axis_name="s")
    z128 = jnp.zeros((125, D), jnp.float32)
    z16 = jnp.zeros((125, 16), jnp.float32)
    ones = jnp.concatenate(
        [jnp.ones((CH, 1), jnp.float32), jnp.zeros((CH, 15), jnp.float32)], 1)
    fn = pl.kernel(
        _sc_body,
        out_type=(
            jax.ShapeDtypeStruct((NC, N, D), jnp.float32),
            jax.ShapeDtypeStruct((NC, N, 16), jnp.float32),
        ),
        mesh=mesh,
        scratch_types=[
            pltpu.VMEM((NCH, CH), jnp.int32),
            pltpu.VMEM((NCH, CH), jnp.int32),
            pltpu.VMEM((CH, D), jnp.float32),
            pltpu.VMEM((CH, D), jnp.float32),
            pltpu.VMEM((CH, 16), jnp.float32),
            pltpu.VMEM_SHARED((N, D), jnp.float32),
            pltpu.VMEM_SHARED((N, 16), jnp.float32),
            pltpu.SemaphoreType.DMA,
            pltpu.SemaphoreType.DMA,
        ],
    )
    return fn(xp, flat3, dst3, z128, z16, ones)


def _final_body(x_ref, ms_ref, dg_ref, lw_ref, b_ref, o_ref):
    ms = ms_ref[0] + ms_ref[1]
    deg = jnp.sum(dg_ref[0], axis=-1) + jnp.sum(dg_ref[1], axis=-1)
    deg = jnp.maximum(deg, 1.0)
    h = ms / deg[:, None]
    h = h + jnp.dot(x_ref[...], lw_ref[...], preferred_element_type=jnp.float32)
    h = h + b_ref[...]
    o_ref[...] = jnp.maximum(h, 0.0)


def _finalize(x, msum, deg, loop_w, bias):
    return pl.pallas_call(
        _final_body,
        grid=(N // NBLK,),
        in_specs=[
            pl.BlockSpec((NBLK, D), lambda i: (i, 0)),
            pl.BlockSpec((NC, NBLK, D), lambda i: (0, i, 0)),
            pl.BlockSpec((NC, NBLK, 16), lambda i: (0, i, 0)),
            pl.BlockSpec((D, D), lambda i: (0, 0)),
            pl.BlockSpec((1, D), lambda i: (0, 0)),
        ],
        out_specs=pl.BlockSpec((NBLK, D), lambda i: (i, 0)),
        out_shape=jax.ShapeDtypeStruct((N, D), jnp.float32),
    )(x, msum, deg, loop_w, bias.reshape(1, D))


def kernel(x, edge_index, etypes, W, loop_w, bias, step=0):
    src = edge_index[0].astype(jnp.int32)
    dst = edge_index[1].astype(jnp.int32)
    et = etypes.astype(jnp.int32)
    flat3 = (et * N + src).reshape(NW, NCH, CH)
    dst3 = dst.reshape(NW, NCH, CH)

    xp = _project(x, W).reshape(R * N, D)
    msum, deg = _sc_aggregate(xp, flat3, dst3)
    return _finalize(x, msum, deg, loop_w, bias)


# speedup vs baseline: 4.4495x; 4.4495x over previous
import jax
import jax.numpy as jnp
from jax import lax
from jax.experimental import pallas as pl
from jax.experimental.pallas import tpu as pltpu
from jax.experimental.pallas import tpu_sc as plsc

N = 10000
E = 320000
D = 128
R = 24

NC = 2
NS = 16
NW = NC * NS
EPW = E // NW
CH = 100
NCH = EPW // CH
STRIPE = N // NS
NBLK = 1000


def _proj_body(x_ref, w_ref, o_ref):
    r = pl.program_id(1)
    o_ref[0] = jnp.dot(x_ref[...], w_ref[r], preferred_element_type=jnp.float32)


def _project(x, W):
    return pl.pallas_call(
        _proj_body,
        grid=(N // NBLK, R),
        in_specs=[
            pl.BlockSpec((NBLK, D), lambda i, r: (i, 0)),
            pl.BlockSpec((R, D, D), lambda i, r: (0, 0, 0)),
        ],
        out_specs=pl.BlockSpec((1, NBLK, D), lambda i, r: (r, i, 0)),
        out_shape=jax.ShapeDtypeStruct((R, N, D), jnp.float32),
    )(x, W)


def _deg_body(dst2_hbm, deg_out, dstv, degloc):
    c = lax.axis_index("c")
    s = lax.axis_index("s")
    wid = c * NS + s

    pltpu.sync_copy(dst2_hbm.at[wid], dstv)

    zeros16 = jnp.zeros((16,), jnp.float32)
    ones16 = jnp.ones((16,), jnp.float32)

    def dzero(i, carry):
        degloc[pl.ds(i * 16, 16)] = zeros16
        return carry

    lax.fori_loop(0, N // 16, dzero, 0)

    def dcount(i, carry):
        plsc.addupdate_scatter(degloc, [dstv[i, :]], ones16)
        return carry

    lax.fori_loop(0, EPW // 16, dcount, 0)
    pltpu.sync_copy(degloc, deg_out.at[wid])


def _sc_deg(dst2):
    mesh = plsc.VectorSubcoreMesh(core_axis_name="c", subcore_axis_name="s")
    fn = pl.kernel(
        _deg_body,
        out_type=jax.ShapeDtypeStruct((NW, N), jnp.float32),
        mesh=mesh,
        scratch_types=[
            pltpu.VMEM((EPW // 16, 16), jnp.int32),
            pltpu.VMEM((N,), jnp.float32),
        ],
        compiler_params=pltpu.CompilerParams(
            use_tc_tiling_on_sc=False, needs_layout_passes=False),
    )
    return fn(dst2)


def _sc_body(xp_hbm, flat_hbm, dst_hbm, z128_hbm,
             msum_out,
             gidx, sidx, rows0, rows1, msum_sh, sem0, sem1):
    c = lax.axis_index("c")
    s = lax.axis_index("s")
    wid = c * NS + s

    for b in range(STRIPE // 125):
        off = s * STRIPE + b * 125
        pltpu.sync_copy(z128_hbm, msum_sh.at[pl.ds(off, 125)])

    pltpu.sync_copy(flat_hbm.at[wid], gidx)
    pltpu.sync_copy(dst_hbm.at[wid], sidx)

    plsc.subcore_barrier()

    def pair(t, carry):
        j0 = t * 2
        j1 = j0 + 1
        cp0 = pltpu.async_copy(xp_hbm.at[gidx.at[j0]], rows0, sem0)
        cp1 = pltpu.async_copy(xp_hbm.at[gidx.at[j1]], rows1, sem1)
        cp0.wait()
        pltpu.sync_copy(rows0, msum_sh.at[sidx.at[j0]], add=True)
        cp1.wait()
        pltpu.sync_copy(rows1, msum_sh.at[sidx.at[j1]], add=True)
        return carry

    lax.fori_loop(0, NCH // 2, pair, 0)

    plsc.subcore_barrier()

    for b in range(STRIPE // 125):
        off = s * STRIPE + b * 125
        pltpu.sync_copy(msum_sh.at[pl.ds(off, 125)],
                        msum_out.at[c, pl.ds(off, 125)])


def _sc_aggregate(xp, flat3, dst3):
    mesh = plsc.VectorSubcoreMesh(core_axis_name="c", subcore_axis_name="s")
    z128 = jnp.zeros((125, D), jnp.float32)
    fn = pl.kernel(
        _sc_body,
        out_type=jax.ShapeDtypeStruct((NC, N, D), jnp.float32),
        mesh=mesh,
        scratch_types=[
            pltpu.VMEM((NCH, CH), jnp.int32),
            pltpu.VMEM((NCH, CH), jnp.int32),
            pltpu.VMEM((CH, D), jnp.float32),
            pltpu.VMEM((CH, D), jnp.float32),
            pltpu.VMEM_SHARED((N, D), jnp.float32),
            pltpu.SemaphoreType.DMA,
            pltpu.SemaphoreType.DMA,
        ],
        compiler_params=pltpu.CompilerParams(
            use_tc_tiling_on_sc=False, needs_layout_passes=False),
    )
    return fn(xp, flat3, dst3, z128)


def _final_body(x_ref, ms_ref, dg_ref, lw_ref, b_ref, o_ref):
    ms = ms_ref[0] + ms_ref[1]
    deg = jnp.sum(dg_ref[...], axis=1)
    deg = jnp.maximum(deg, 1.0)
    h = ms / deg[:, None]
    h = h + jnp.dot(x_ref[...], lw_ref[...], preferred_element_type=jnp.float32)
    h = h + b_ref[...]
    o_ref[...] = jnp.maximum(h, 0.0)


def _finalize(x, msum, deg, loop_w, bias):
    return pl.pallas_call(
        _final_body,
        grid=(N // NBLK,),
        in_specs=[
            pl.BlockSpec((NBLK, D), lambda i: (i, 0)),
            pl.BlockSpec((NC, NBLK, D), lambda i: (0, i, 0)),
            pl.BlockSpec((NBLK, NW), lambda i: (i, 0)),
            pl.BlockSpec((D, D), lambda i: (0, 0)),
            pl.BlockSpec((1, D), lambda i: (0, 0)),
        ],
        out_specs=pl.BlockSpec((NBLK, D), lambda i: (i, 0)),
        out_shape=jax.ShapeDtypeStruct((N, D), jnp.float32),
    )(x, msum, deg, loop_w, bias.reshape(1, D))


def kernel(x, edge_index, etypes, W, loop_w, bias, step=0):
    src = edge_index[0].astype(jnp.int32)
    dst = edge_index[1].astype(jnp.int32)
    et = etypes.astype(jnp.int32)
    flat3 = (et * N + src).reshape(NW, NCH, CH)
    dst3 = dst.reshape(NW, NCH, CH)
    dst2 = dst.reshape(NW, EPW // 16, 16)

    deg = _sc_deg(dst2)
    xp = _project(x, W).reshape(R * N, D)
    msum = _sc_aggregate(xp, flat3, dst3)
    return _finalize(x, msum, deg.T, loop_w, bias)


# speedup vs baseline: 4.6073x; 1.0354x over previous
import jax
import jax.numpy as jnp
from jax import lax
from jax.experimental import pallas as pl
from jax.experimental.pallas import tpu as pltpu
from jax.experimental.pallas import tpu_sc as plsc

N = 10000
E = 320000
D = 128
R = 24

NC = 2
NS = 16
NW = NC * NS
EPW = E // NW
CH = 50
NCH = EPW // CH
STRIPE = N // NS
NBLK = 1000


def _proj_body(x_ref, w_ref, o_ref):
    o_ref[...] = jnp.dot(x_ref[...], w_ref[...],
                         preferred_element_type=jnp.float32)


def _project(x, W2):
    blk = 400
    return pl.pallas_call(
        _proj_body,
        grid=(N // blk,),
        in_specs=[
            pl.BlockSpec((blk, D), lambda i: (i, 0)),
            pl.BlockSpec((D, R * D), lambda i: (0, 0)),
        ],
        out_specs=pl.BlockSpec((blk, R * D), lambda i: (i, 0)),
        out_shape=jax.ShapeDtypeStruct((N, R * D), jnp.float32),
    )(x, W2)


def _deg_body(dst2_hbm, deg_out, dstv, degloc):
    c = lax.axis_index("c")
    s = lax.axis_index("s")
    wid = c * NS + s

    pltpu.sync_copy(dst2_hbm.at[wid], dstv)

    zeros16 = jnp.zeros((16,), jnp.float32)
    ones16 = jnp.ones((16,), jnp.float32)

    def dzero(i, carry):
        degloc[pl.ds(i * 16, 16)] = zeros16
        return carry

    lax.fori_loop(0, N // 16, dzero, 0)

    def dcount(i, carry):
        plsc.addupdate_scatter(degloc, [dstv[i, :]], ones16)
        return carry

    lax.fori_loop(0, EPW // 16, dcount, 0)
    pltpu.sync_copy(degloc, deg_out.at[wid])


def _sc_deg(dst2):
    mesh = plsc.VectorSubcoreMesh(core_axis_name="c", subcore_axis_name="s")
    fn = pl.kernel(
        _deg_body,
        out_type=jax.ShapeDtypeStruct((NW, N), jnp.float32),
        mesh=mesh,
        scratch_types=[
            pltpu.VMEM((EPW // 16, 16), jnp.int32),
            pltpu.VMEM((N,), jnp.float32),
        ],
        compiler_params=pltpu.CompilerParams(
            use_tc_tiling_on_sc=False, needs_layout_passes=False),
    )
    return fn(dst2)


def _sc_body(xp_hbm, flat_hbm, dst_hbm, z128_hbm,
             msum_out,
             gidx, sidx, rows0, rows1, rows2, rows3, msum_sh,
             gs0, gs1, gs2, gs3, ss0, ss1, ss2, ss3):
    c = lax.axis_index("c")
    s = lax.axis_index("s")
    wid = c * NS + s

    for b in range(STRIPE // 125):
        off = s * STRIPE + b * 125
        pltpu.sync_copy(z128_hbm, msum_sh.at[pl.ds(off, 125)])

    pltpu.sync_copy(flat_hbm.at[wid], gidx)
    pltpu.sync_copy(dst_hbm.at[wid], sidx)

    plsc.subcore_barrier()

    rows = (rows0, rows1, rows2, rows3)
    gs = (gs0, gs1, gs2, gs3)
    ss = (ss0, ss1, ss2, ss3)

    def quad(t, carry):
        j = t * 4
        cps = [
            pltpu.async_copy(xp_hbm.at[gidx.at[j + b]], rows[b], gs[b])
            for b in range(4)
        ]
        scs = []
        for b in range(4):
            cps[b].wait()
            scs.append(pltpu.async_copy(
                rows[b], msum_sh.at[sidx.at[j + b]], ss[b], add=True))
        for b in range(4):
            scs[b].wait()
        return carry

    lax.fori_loop(0, NCH // 4, quad, 0)

    plsc.subcore_barrier()

    for b in range(STRIPE // 125):
        off = s * STRIPE + b * 125
        pltpu.sync_copy(msum_sh.at[pl.ds(off, 125)],
                        msum_out.at[c, pl.ds(off, 125)])


def _sc_aggregate(xp, flat3, dst3):
    mesh = plsc.VectorSubcoreMesh(core_axis_name="c", subcore_axis_name="s")
    z128 = jnp.zeros((125, D), jnp.float32)
    fn = pl.kernel(
        _sc_body,
        out_type=jax.ShapeDtypeStruct((NC, N, D), jnp.float32),
        mesh=mesh,
        scratch_types=[
            pltpu.VMEM((NCH, CH), jnp.int32),
            pltpu.VMEM((NCH, CH), jnp.int32),
            pltpu.VMEM((CH, D), jnp.float32),
            pltpu.VMEM((CH, D), jnp.float32),
            pltpu.VMEM((CH, D), jnp.float32),
            pltpu.VMEM((CH, D), jnp.float32),
            pltpu.VMEM_SHARED((N, D), jnp.float32),
        ] + [pltpu.SemaphoreType.DMA] * 8,
        compiler_params=pltpu.CompilerParams(
            use_tc_tiling_on_sc=False, needs_layout_passes=False),
    )
    return fn(xp, flat3, dst3, z128)


def _final_body(x_ref, ms_ref, dg_ref, lw_ref, b_ref, o_ref):
    ms = ms_ref[0] + ms_ref[1]
    deg = jnp.sum(dg_ref[...], axis=1)
    deg = jnp.maximum(deg, 1.0)
    h = ms / deg[:, None]
    h = h + jnp.dot(x_ref[...], lw_ref[...], preferred_element_type=jnp.float32)
    h = h + b_ref[...]
    o_ref[...] = jnp.maximum(h, 0.0)


def _finalize(x, msum, deg, loop_w, bias):
    return pl.pallas_call(
        _final_body,
        grid=(N // NBLK,),
        in_specs=[
            pl.BlockSpec((NBLK, D), lambda i: (i, 0)),
            pl.BlockSpec((NC, NBLK, D), lambda i: (0, i, 0)),
            pl.BlockSpec((NBLK, NW), lambda i: (i, 0)),
            pl.BlockSpec((D, D), lambda i: (0, 0)),
            pl.BlockSpec((1, D), lambda i: (0, 0)),
        ],
        out_specs=pl.BlockSpec((NBLK, D), lambda i: (i, 0)),
        out_shape=jax.ShapeDtypeStruct((N, D), jnp.float32),
    )(x, msum, deg, loop_w, bias.reshape(1, D))


def kernel(x, edge_index, etypes, W, loop_w, bias, step=0):
    src = edge_index[0].astype(jnp.int32)
    dst = edge_index[1].astype(jnp.int32)
    et = etypes.astype(jnp.int32)
    flat3 = (src * R + et).reshape(NW, NCH, CH)
    dst3 = dst.reshape(NW, NCH, CH)
    dst2 = dst.reshape(NW, EPW // 16, 16)
    W2 = jnp.transpose(W, (1, 0, 2)).reshape(D, R * D)

    deg = _sc_deg(dst2)
    xp = _project(x, W2).reshape(N * R, D)
    msum = _sc_aggregate(xp, flat3, dst3)
    return _finalize(x, msum, deg.T, loop_w, bias)


# speedup vs baseline: 5.8066x; 1.2603x over previous
import jax
import jax.numpy as jnp
from jax import lax
from jax.experimental import pallas as pl
from jax.experimental.pallas import tpu as pltpu
from jax.experimental.pallas import tpu_sc as plsc

N = 10000
E = 320000
D = 128
R = 24

NC = 2
NS = 16
NW = NC * NS
EPW = E // NW
CH = 50
NCH = EPW // CH
STRIPE = N // NS
NBLK = 1000


def _proj_body(x_ref, w_ref, o_ref):
    o_ref[0] = jnp.dot(x_ref[...], w_ref[0],
                       preferred_element_type=jnp.float32)


def _project(x, W):
    return pl.pallas_call(
        _proj_body,
        grid=(R,),
        in_specs=[
            pl.BlockSpec((N, D), lambda r: (0, 0)),
            pl.BlockSpec((1, D, D), lambda r: (r, 0, 0)),
        ],
        out_specs=pl.BlockSpec((1, N, D), lambda r: (r, 0, 0)),
        out_shape=jax.ShapeDtypeStruct((R, N, D), jnp.float32),
    )(x, W)


def _deg_body(dst2_hbm, deg_out, dstv, degloc):
    c = lax.axis_index("c")
    s = lax.axis_index("s")
    wid = c * NS + s

    pltpu.sync_copy(dst2_hbm.at[wid], dstv)

    zeros16 = jnp.zeros((16,), jnp.float32)
    ones16 = jnp.ones((16,), jnp.float32)

    def dzero(i, carry):
        degloc[pl.ds(i * 16, 16)] = zeros16
        return carry

    lax.fori_loop(0, N // 16, dzero, 0)

    def dcount(i, carry):
        plsc.addupdate_scatter(degloc, [dstv[i, :]], ones16)
        return carry

    lax.fori_loop(0, EPW // 16, dcount, 0)
    pltpu.sync_copy(degloc, deg_out.at[wid])


def _sc_deg(dst2):
    mesh = plsc.VectorSubcoreMesh(core_axis_name="c", subcore_axis_name="s")
    fn = pl.kernel(
        _deg_body,
        out_type=jax.ShapeDtypeStruct((NW, N), jnp.float32),
        mesh=mesh,
        scratch_types=[
            pltpu.VMEM((EPW // 16, 16), jnp.int32),
            pltpu.VMEM((N,), jnp.float32),
        ],
        compiler_params=pltpu.CompilerParams(
            use_tc_tiling_on_sc=False, needs_layout_passes=False),
    )
    return fn(dst2)


def _sc_body(xp_hbm, flat_hbm, dst_hbm, z128_hbm,
             msum_out,
             gidx, sidx, rows0, rows1, rows2, rows3, msum_sh,
             gs0, gs1, gs2, gs3, ss0, ss1, ss2, ss3):
    c = lax.axis_index("c")
    s = lax.axis_index("s")
    wid = c * NS + s

    for b in range(STRIPE // 125):
        off = s * STRIPE + b * 125
        pltpu.sync_copy(z128_hbm, msum_sh.at[pl.ds(off, 125)])

    pltpu.sync_copy(flat_hbm.at[wid], gidx)
    pltpu.sync_copy(dst_hbm.at[wid], sidx)

    plsc.subcore_barrier()

    rows = (rows0, rows1, rows2, rows3)
    gs = (gs0, gs1, gs2, gs3)
    ss = (ss0, ss1, ss2, ss3)

    def quad(t, carry):
        j = t * 4
        cps = [
            pltpu.async_copy(xp_hbm.at[gidx.at[j + b]], rows[b], gs[b])
            for b in range(4)
        ]
        scs = []
        for b in range(4):
            cps[b].wait()
            scs.append(pltpu.async_copy(
                rows[b], msum_sh.at[sidx.at[j + b]], ss[b], add=True))
        for b in range(4):
            scs[b].wait()
        return carry

    lax.fori_loop(0, NCH // 4, quad, 0)

    plsc.subcore_barrier()

    for b in range(STRIPE // 125):
        off = s * STRIPE + b * 125
        pltpu.sync_copy(msum_sh.at[pl.ds(off, 125)],
                        msum_out.at[c, pl.ds(off, 125)])


def _sc_aggregate(xp, flat3, dst3):
    mesh = plsc.VectorSubcoreMesh(core_axis_name="c", subcore_axis_name="s")
    z128 = jnp.zeros((125, D), jnp.float32)
    fn = pl.kernel(
        _sc_body,
        out_type=jax.ShapeDtypeStruct((NC, N, D), jnp.float32),
        mesh=mesh,
        scratch_types=[
            pltpu.VMEM((NCH, CH), jnp.int32),
            pltpu.VMEM((NCH, CH), jnp.int32),
            pltpu.VMEM((CH, D), jnp.float32),
            pltpu.VMEM((CH, D), jnp.float32),
            pltpu.VMEM((CH, D), jnp.float32),
            pltpu.VMEM((CH, D), jnp.float32),
            pltpu.VMEM_SHARED((N, D), jnp.float32),
        ] + [pltpu.SemaphoreType.DMA] * 8,
        compiler_params=pltpu.CompilerParams(
            use_tc_tiling_on_sc=False, needs_layout_passes=False),
    )
    return fn(xp, flat3, dst3, z128)


def _final_body(x_ref, ms_ref, dg_ref, lw_ref, b_ref, o_ref):
    ms = ms_ref[0] + ms_ref[1]
    deg = jnp.sum(dg_ref[...], axis=1)
    deg = jnp.maximum(deg, 1.0)
    h = ms / deg[:, None]
    h = h + jnp.dot(x_ref[...], lw_ref[...], preferred_element_type=jnp.float32)
    h = h + b_ref[...]
    o_ref[...] = jnp.maximum(h, 0.0)


def _finalize(x, msum, deg, loop_w, bias):
    return pl.pallas_call(
        _final_body,
        grid=(N // NBLK,),
        in_specs=[
            pl.BlockSpec((NBLK, D), lambda i: (i, 0)),
            pl.BlockSpec((NC, NBLK, D), lambda i: (0, i, 0)),
            pl.BlockSpec((NBLK, NW), lambda i: (i, 0)),
            pl.BlockSpec((D, D), lambda i: (0, 0)),
            pl.BlockSpec((1, D), lambda i: (0, 0)),
        ],
        out_specs=pl.BlockSpec((NBLK, D), lambda i: (i, 0)),
        out_shape=jax.ShapeDtypeStruct((N, D), jnp.float32),
    )(x, msum, deg, loop_w, bias.reshape(1, D))


def kernel(x, edge_index, etypes, W, loop_w, bias, step=0):
    src = edge_index[0].astype(jnp.int32)
    dst = edge_index[1].astype(jnp.int32)
    et = etypes.astype(jnp.int32)
    flat3 = (et * N + src).reshape(NW, NCH, CH)
    dst3 = dst.reshape(NW, NCH, CH)
    dst2 = dst.reshape(NW, EPW // 16, 16)

    deg = _sc_deg(dst2)
    xp = _project(x, W).reshape(R * N, D)
    msum = _sc_aggregate(xp, flat3, dst3)
    return _finalize(x, msum, deg.T, loop_w, bias)


# speedup vs baseline: 6.6674x; 1.1482x over previous
import jax
import jax.numpy as jnp
from jax import lax
from jax.experimental import pallas as pl
from jax.experimental.pallas import tpu as pltpu
from jax.experimental.pallas import tpu_sc as plsc

N = 10000
E = 320000
D = 128
R = 24

NC = 2
NS = 16
NW = NC * NS
EPW = E // NW
CH = 50
NCH = EPW // CH
STRIPE = N // NS
NBLK = 1000


def _proj_body(x_ref, w_ref, o_ref):
    o_ref[0] = jnp.dot(x_ref[...], w_ref[0],
                       preferred_element_type=jnp.float32)


def _project(x, W):
    return pl.pallas_call(
        _proj_body,
        grid=(R,),
        in_specs=[
            pl.BlockSpec((N, D), lambda r: (0, 0)),
            pl.BlockSpec((1, D, D), lambda r: (r, 0, 0)),
        ],
        out_specs=pl.BlockSpec((1, N, D), lambda r: (r, 0, 0)),
        out_shape=jax.ShapeDtypeStruct((R, N, D), jnp.float32),
    )(x, W)


def _deg_body(dst2_hbm, deg_out, dstv, degloc):
    c = lax.axis_index("c")
    s = lax.axis_index("s")
    wid = c * NS + s

    pltpu.sync_copy(dst2_hbm.at[wid], dstv)

    zeros16 = jnp.zeros((16,), jnp.float32)
    ones16 = jnp.ones((16,), jnp.float32)

    def dzero(i, carry):
        degloc[pl.ds(i * 16, 16)] = zeros16
        return carry

    lax.fori_loop(0, N // 16, dzero, 0)

    def dcount(i, carry):
        plsc.addupdate_scatter(degloc, [dstv[i, :]], ones16)
        return carry

    lax.fori_loop(0, EPW // 16, dcount, 0)
    pltpu.sync_copy(degloc, deg_out.at[wid])


def _sc_deg(dst2):
    mesh = plsc.VectorSubcoreMesh(core_axis_name="c", subcore_axis_name="s")
    fn = pl.kernel(
        _deg_body,
        out_type=jax.ShapeDtypeStruct((NW, N), jnp.float32),
        mesh=mesh,
        scratch_types=[
            pltpu.VMEM((EPW // 16, 16), jnp.int32),
            pltpu.VMEM((N,), jnp.float32),
        ],
        compiler_params=pltpu.CompilerParams(
            use_tc_tiling_on_sc=False, needs_layout_passes=False),
    )
    return fn(dst2)


def _sc_body(xp_hbm, flat_hbm, dst_hbm, z128_hbm,
             msum_out,
             gidx, sidx, rows0, rows1, rows2, rows3, msum_sh,
             gs0, gs1, gs2, gs3, ss0, ss1, ss2, ss3):
    c = lax.axis_index("c")
    s = lax.axis_index("s")
    wid = c * NS + s

    for b in range(STRIPE // 125):
        off = s * STRIPE + b * 125
        pltpu.sync_copy(z128_hbm, msum_sh.at[pl.ds(off, 125)])

    pltpu.sync_copy(flat_hbm.at[wid], gidx)
    pltpu.sync_copy(dst_hbm.at[wid], sidx)

    plsc.subcore_barrier()

    rows = (rows0, rows1, rows2, rows3)
    gs = (gs0, gs1, gs2, gs3)
    ss = (ss0, ss1, ss2, ss3)
    niter = NCH // 4

    for b in range(4):
        pltpu.async_copy(xp_hbm.at[gidx.at[b]], rows[b], gs[b])

    def step(t, carry):
        j = t * 4
        for b in range(4):
            pltpu.make_async_copy(
                xp_hbm.at[pl.ds(0, CH)], rows[b], gs[b]).wait()
            pltpu.async_copy(rows[b], msum_sh.at[sidx.at[j + b]], ss[b],
                             add=True)
        for b in range(4):
            pltpu.make_async_copy(
                xp_hbm.at[pl.ds(0, CH)], rows[b], ss[b]).wait()

            @pl.when(t + 1 < niter)
            def _():
                pltpu.async_copy(
                    xp_hbm.at[gidx.at[j + 4 + b]], rows[b], gs[b])
        return carry

    lax.fori_loop(0, niter, step, 0)

    plsc.subcore_barrier()

    for b in range(STRIPE // 125):
        off = s * STRIPE + b * 125
        pltpu.sync_copy(msum_sh.at[pl.ds(off, 125)],
                        msum_out.at[c, pl.ds(off, 125)])


def _sc_aggregate(xp, flat3, dst3):
    mesh = plsc.VectorSubcoreMesh(core_axis_name="c", subcore_axis_name="s")
    z128 = jnp.zeros((125, D), jnp.float32)
    fn = pl.kernel(
        _sc_body,
        out_type=jax.ShapeDtypeStruct((NC, N, D), jnp.float32),
        mesh=mesh,
        scratch_types=[
            pltpu.VMEM((NCH, CH), jnp.int32),
            pltpu.VMEM((NCH, CH), jnp.int32),
            pltpu.VMEM((CH, D), jnp.float32),
            pltpu.VMEM((CH, D), jnp.float32),
            pltpu.VMEM((CH, D), jnp.float32),
            pltpu.VMEM((CH, D), jnp.float32),
            pltpu.VMEM_SHARED((N, D), jnp.float32),
        ] + [pltpu.SemaphoreType.DMA] * 8,
        compiler_params=pltpu.CompilerParams(
            use_tc_tiling_on_sc=False, needs_layout_passes=False),
    )
    return fn(xp, flat3, dst3, z128)


def _final_body(x_ref, ms_ref, dg_ref, lw_ref, b_ref, o_ref):
    ms = ms_ref[0] + ms_ref[1]
    deg = jnp.sum(dg_ref[...], axis=1)
    deg = jnp.maximum(deg, 1.0)
    h = ms / deg[:, None]
    h = h + jnp.dot(x_ref[...], lw_ref[...], preferred_element_type=jnp.float32)
    h = h + b_ref[...]
    o_ref[...] = jnp.maximum(h, 0.0)


def _finalize(x, msum, deg, loop_w, bias):
    return pl.pallas_call(
        _final_body,
        grid=(N // NBLK,),
        in_specs=[
            pl.BlockSpec((NBLK, D), lambda i: (i, 0)),
            pl.BlockSpec((NC, NBLK, D), lambda i: (0, i, 0)),
            pl.BlockSpec((NBLK, NW), lambda i: (i, 0)),
            pl.BlockSpec((D, D), lambda i: (0, 0)),
            pl.BlockSpec((1, D), lambda i: (0, 0)),
        ],
        out_specs=pl.BlockSpec((NBLK, D), lambda i: (i, 0)),
        out_shape=jax.ShapeDtypeStruct((N, D), jnp.float32),
    )(x, msum, deg, loop_w, bias.reshape(1, D))


def kernel(x, edge_index, etypes, W, loop_w, bias, step=0):
    src = edge_index[0].astype(jnp.int32)
    dst = edge_index[1].astype(jnp.int32)
    et = etypes.astype(jnp.int32)
    flat3 = (et * N + src).reshape(NW, NCH, CH)
    dst3 = dst.reshape(NW, NCH, CH)
    dst2 = dst.reshape(NW, EPW // 16, 16)

    deg = _sc_deg(dst2)
    xp = _project(x, W).reshape(R * N, D)
    msum = _sc_aggregate(xp, flat3, dst3)
    return _finalize(x, msum, deg.T, loop_w, bias)
